# baseline (speedup 1.0000x reference)
import functools

import jax
import jax.numpy as jnp
from jax import lax
from jax.experimental import pallas as pl
from jax.experimental.pallas import tpu as pltpu
from jax.experimental.pallas import tpu_sc as plsc

N = 10000
E = 320000
E2 = E + N
IN_CH = 128
HID = 64
OUT_CH = 64
HEADS = 4
ALPHA = 0.5
AD = 8

NC = 2
NS = 16
NW = NC * NS
K = 128
_NCH = -(-E2 // (NW * K))
_NCH += _NCH % 2
E2P = NW * K * _NCH
EPW = E2P // NW
NG = EPW // K
NP = N + 16


def _sc_gather(d):
    mesh = plsc.VectorSubcoreMesh(core_axis_name="c", subcore_axis_name="s")

    @functools.partial(
        pl.kernel, mesh=mesh,
        out_type=jax.ShapeDtypeStruct((E2P, d), jnp.float32),
        scratch_types=[
            pltpu.VMEM((K,), jnp.int32),
            pltpu.VMEM((K,), jnp.int32),
            pltpu.VMEM((K, d), jnp.float32),
            pltpu.VMEM((K, d), jnp.float32),
            pltpu.SemaphoreType.DMA,
            pltpu.SemaphoreType.DMA,
            pltpu.SemaphoreType.DMA,
            pltpu.SemaphoreType.DMA,
        ],
        compiler_params=pltpu.CompilerParams(use_tc_tiling_on_sc=False),
    )
    def k(table_hbm, idx_hbm, out_hbm, idx_a, idx_b, rows_a, rows_b,
          sem_a, sem_b, wsem_a, wsem_b):
        wid = lax.axis_index("s") * NC + lax.axis_index("c")

        def body(g2, carry):
            base = wid * EPW + 2 * g2 * K
            pltpu.sync_copy(idx_hbm.at[pl.ds(base, K)], idx_a)
            pltpu.sync_copy(idx_hbm.at[pl.ds(base + K, K)], idx_b)
            ga = pltpu.async_copy(table_hbm.at[idx_a], rows_a, sem_a)
            gb = pltpu.async_copy(table_hbm.at[idx_b], rows_b, sem_b)
            ga.wait()
            wa = pltpu.async_copy(rows_a, out_hbm.at[pl.ds(base, K)], wsem_a)
            gb.wait()
            wb = pltpu.async_copy(rows_b, out_hbm.at[pl.ds(base + K, K)],
                                  wsem_b)
            wa.wait()
            wb.wait()
            return carry

        lax.fori_loop(0, NG // 2, body, 0)

    return k


def _sc_scatter_add(d):
    mesh = plsc.VectorSubcoreMesh(core_axis_name="c", subcore_axis_name="s")
    rpt = NP // NS

    @functools.partial(
        pl.kernel, mesh=mesh,
        out_type=jax.ShapeDtypeStruct((NC, NP, d), jnp.float32),
        scratch_types=[
            pltpu.VMEM((K,), jnp.int32),
            pltpu.VMEM((K,), jnp.int32),
            pltpu.VMEM((K, d), jnp.float32),
            pltpu.VMEM((K, d), jnp.float32),
            pltpu.VMEM_SHARED((NP, d), jnp.float32),
            pltpu.SemaphoreType.DMA,
            pltpu.SemaphoreType.DMA,
        ],
        compiler_params=pltpu.CompilerParams(use_tc_tiling_on_sc=False),
    )
    def k(vals_hbm, idx_hbm, zeros_hbm, out_hbm, idx_a, idx_b, vals_a,
          vals_b, acc_sh, sem_a, sem_b):
        cid = lax.axis_index("c")
        sid = lax.axis_index("s")
        wid = sid * NC + cid
        pltpu.sync_copy(zeros_hbm.at[pl.ds(sid * rpt, rpt)],
                        acc_sh.at[pl.ds(sid * rpt, rpt)])
        plsc.subcore_barrier()

        def body(g2, carry):
            base = wid * EPW + 2 * g2 * K
            ia = pltpu.async_copy(idx_hbm.at[pl.ds(base, K)], idx_a, sem_a)
            va = pltpu.async_copy(vals_hbm.at[pl.ds(base, K)], vals_a, sem_a)
            ib = pltpu.async_copy(idx_hbm.at[pl.ds(base + K, K)], idx_b,
                                  sem_b)
            vb = pltpu.async_copy(vals_hbm.at[pl.ds(base + K, K)], vals_b,
                                  sem_b)
            ia.wait()
            va.wait()
            pltpu.sync_copy(vals_a, acc_sh.at[idx_a], add=True)
            ib.wait()
            vb.wait()
            pltpu.sync_copy(vals_b, acc_sh.at[idx_b], add=True)
            return carry

        lax.fori_loop(0, NG // 2, body, 0)
        plsc.subcore_barrier()
        pltpu.sync_copy(acc_sh.at[pl.ds(sid * rpt, rpt)],
                        out_hbm.at[cid, pl.ds(sid * rpt, rpt)])

    return k


def _elu(v):
    return jnp.where(v > 0, v, jnp.exp(jnp.minimum(v, 0.0)) - 1.0)

def _proj_call(heads, hid, din):
    bn = 400
    w = heads * hid

    def body(x_ref, w_ref, asrc_ref, adst_ref, h_ref, as_ref, ad_ref):
        h = jnp.dot(x_ref[...], w_ref[...], preferred_element_type=jnp.float32)
        h_ref[...] = h
        h4 = h.reshape(bn, heads, hid)
        a_s = (h4 * asrc_ref[...][:heads][None]).sum(-1)
        a_d = (h4 * adst_ref[...][:heads][None]).sum(-1)
        pad = jnp.zeros((bn, AD - heads), jnp.float32)
        as_ref[...] = jnp.concatenate([a_s, pad], axis=1)
        ad_ref[...] = jnp.concatenate([a_d, pad], axis=1)

    return pl.pallas_call(
        body,
        grid=(N // bn,),
        in_specs=[
            pl.BlockSpec((bn, din), lambda i: (i, 0)),
            pl.BlockSpec((din, w), lambda i: (0, 0)),
            pl.BlockSpec((8, hid), lambda i: (0, 0)),
            pl.BlockSpec((8, hid), lambda i: (0, 0)),
        ],
        out_specs=[
            pl.BlockSpec((bn, w), lambda i: (i, 0)),
            pl.BlockSpec((bn, AD), lambda i: (i, 0)),
            pl.BlockSpec((bn, AD), lambda i: (i, 0)),
        ],
        out_shape=[
            jax.ShapeDtypeStruct((N, w), jnp.float32),
            jax.ShapeDtypeStruct((N, AD), jnp.float32),
            jax.ShapeDtypeStruct((N, AD), jnp.float32),
        ],
    )


_EW_R = E2P * AD // 128


def _eexp_call():
    br = 256

    def body(s_ref, d_ref, o_ref):
        v = s_ref[...] + d_ref[...]
        v = jnp.where(v >= 0, v, 0.2 * v)
        o_ref[...] = jnp.exp(v)

    return pl.pallas_call(
        body,
        grid=(_EW_R // br,),
        in_specs=[pl.BlockSpec((br, 128), lambda i: (i, 0))] * 2,
        out_specs=pl.BlockSpec((br, 128), lambda i: (i, 0)),
        out_shape=jax.ShapeDtypeStruct((_EW_R, 128), jnp.float32),
    )


def _alpha_call():
    br = 256

    def body(e_ref, dn_ref, o_ref):
        o_ref[...] = e_ref[...] / (dn_ref[...] + 1e-16)

    return pl.pallas_call(
        body,
        grid=(_EW_R // br,),
        in_specs=[pl.BlockSpec((br, 128), lambda i: (i, 0))] * 2,
        out_specs=pl.BlockSpec((br, 128), lambda i: (i, 0)),
        out_shape=jax.ShapeDtypeStruct((_EW_R, 128), jnp.float32),
    )


def _densum_call(d):
    def body(p_ref, o_ref):
        o_ref[...] = p_ref[0] + p_ref[1]

    return pl.pallas_call(
        body,
        grid=(1,),
        in_specs=[pl.BlockSpec((2, NP, d), lambda i: (0, 0, 0))],
        out_specs=pl.BlockSpec((NP, d), lambda i: (0, 0)),
        out_shape=jax.ShapeDtypeStruct((NP, d), jnp.float32),
    )


def _msg_call(heads, hid):
    bm = 512
    w = heads * hid
    nout = max(1, w // 128)
    wo = w // nout

    def body(h_ref, a_ref, *o_refs):
        hh = h_ref[...].reshape(bm, heads, hid)
        al = a_ref[...][:, :heads]
        m = (hh * al[:, :, None]).reshape(bm, w)
        for j, o_ref in enumerate(o_refs):
            o_ref[...] = m[:, j * wo:(j + 1) * wo]

    return pl.pallas_call(
        body,
        grid=(E2P // bm,),
        in_specs=[
            pl.BlockSpec((bm, w), lambda i: (i, 0)),
            pl.BlockSpec((bm, AD), lambda i: (i, 0)),
        ],
        out_specs=[pl.BlockSpec((bm, wo), lambda i: (i, 0))] * nout,
        out_shape=[jax.ShapeDtypeStruct((E2P, wo), jnp.float32)] * nout,
    )


def _fuse_call():
    bn = 400

    def body(sp0_ref, sp1_ref, ge0_ref, ge1_ref, bsp_ref, bge_ref, o_ref):
        sp = jnp.concatenate([sp0_ref[0] + sp0_ref[1],
                              sp1_ref[0] + sp1_ref[1]], axis=1)
        ge = jnp.concatenate([ge0_ref[0] + ge0_ref[1],
                              ge1_ref[0] + ge1_ref[1]], axis=1)
        sp = _elu(sp + bsp_ref[...][0][None])
        ge = _elu(ge + bge_ref[...][0][None])
        o_ref[...] = ALPHA * ge + (1.0 - ALPHA) * sp

    w = HEADS * HID
    return pl.pallas_call(
        body,
        grid=(N // bn,),
        in_specs=[pl.BlockSpec((2, bn, 128), lambda i: (0, i, 0))] * 4
        + [pl.BlockSpec((8, w), lambda i: (0, 0))] * 2,
        out_specs=pl.BlockSpec((bn, w), lambda i: (i, 0)),
        out_shape=jax.ShapeDtypeStruct((N, w), jnp.float32),
    )


def _dec_call():
    bn = 400

    def body(p_ref, bfu_ref, w1_ref, b1_ref, w2_ref, b2_ref,
             emb_ref, rec_ref):
        emb = p_ref[0] + p_ref[1] + bfu_ref[...][0][None]
        emb_ref[...] = emb
        d1 = _elu(
            jnp.dot(emb, w1_ref[...], preferred_element_type=jnp.float32)
            + b1_ref[...][0][None])
        rec_ref[...] = jax.nn.relu(
            jnp.dot(d1, w2_ref[...], preferred_element_type=jnp.float32)
            + b2_ref[...][0][None])

    return pl.pallas_call(
        body,
        grid=(N // bn,),
        in_specs=[
            pl.BlockSpec((2, bn, OUT_CH), lambda i: (0, i, 0)),
            pl.BlockSpec((8, OUT_CH), lambda i: (0, 0)),
            pl.BlockSpec((OUT_CH, HID), lambda i: (0, 0)),
            pl.BlockSpec((8, HID), lambda i: (0, 0)),
            pl.BlockSpec((HID, IN_CH), lambda i: (0, 0)),
            pl.BlockSpec((8, IN_CH), lambda i: (0, 0)),
        ],
        out_specs=[
            pl.BlockSpec((bn, OUT_CH), lambda i: (i, 0)),
            pl.BlockSpec((bn, IN_CH), lambda i: (i, 0)),
        ],
        out_shape=[
            jax.ShapeDtypeStruct((N, OUT_CH), jnp.float32),
            jax.ShapeDtypeStruct((N, IN_CH), jnp.float32),
        ],
    )


def _pad_rows(a, rows):
    return jnp.concatenate(
        [a, jnp.zeros((rows - a.shape[0],) + a.shape[1:], a.dtype)], axis=0)


def _edge_layer(src, dst, h, a_src_p, a_dst_p, heads, hid, zeros_ad, zeros_h):
    as_e = _sc_gather(AD)(a_src_p, src)
    ad_e = _sc_gather(AD)(a_dst_p, dst)
    e_exp2 = _eexp_call()(as_e.reshape(_EW_R, 128), ad_e.reshape(_EW_R, 128))
    den_p = _sc_scatter_add(AD)(e_exp2.reshape(E2P, AD), dst, zeros_ad)
    den = _densum_call(AD)(den_p)
    dn_e = _sc_gather(AD)(den, dst)
    alpha2 = _alpha_call()(e_exp2, dn_e.reshape(_EW_R, 128))
    alpha_p = alpha2.reshape(E2P, AD)
    h_e = _sc_gather(heads * hid)(h, src)
    msgs = _msg_call(heads, hid)(h_e, alpha_p)
    if not isinstance(msgs, (list, tuple)):
        msgs = [msgs]
    parts = [_sc_scatter_add(m.shape[1])(m, dst, zeros_h[:, :m.shape[1]])
             for m in msgs]
    return alpha_p, parts


def kernel(x, spatial_edge_index, gene_sim_edge_index, W_sp, att_src_sp,
           att_dst_sp, b_sp, W_ge, att_src_ge, att_dst_ge, b_ge, W_fu,
           att_src_fu, att_dst_fu, b_fu, W_d1, b_d1, W_d2, b_d2):
    i32 = jnp.int32
    loop = jnp.arange(N, dtype=i32)
    pad_n = E2P - E2

    def mk_edges(ei):
        src = jnp.concatenate(
            [ei[0].astype(i32), loop, jnp.zeros((pad_n,), i32)])
        dst = jnp.concatenate(
            [ei[1].astype(i32), loop, jnp.full((pad_n,), N, i32)])
        return src, dst

    src_sp, dst_sp = mk_edges(spatial_edge_index)
    src_ge, dst_ge = mk_edges(gene_sim_edge_index)

    zeros_ad = jnp.zeros((NP, AD), jnp.float32)
    zeros_h = jnp.zeros((NP, 128), jnp.float32)

    def pad8(b):
        return jnp.tile(b.reshape(1, -1), (8, 1))

    att8 = lambda a: _pad_rows(a, 8)

    h_sp, as_sp, ads_sp = _proj_call(HEADS, HID, IN_CH)(
        x, W_sp, att8(att_src_sp), att8(att_dst_sp))
    h_ge, as_ge, ads_ge = _proj_call(HEADS, HID, IN_CH)(
        x, W_ge, att8(att_src_ge), att8(att_dst_ge))

    alpha_sp, parts_sp = _edge_layer(
        src_sp, dst_sp, h_sp, _pad_rows(as_sp, NP), _pad_rows(ads_sp, NP),
        HEADS, HID, zeros_ad, zeros_h)
    alpha_ge, parts_ge = _edge_layer(
        src_ge, dst_ge, h_ge, _pad_rows(as_ge, NP), _pad_rows(ads_ge, NP),
        HEADS, HID, zeros_ad, zeros_h)

    h_fused = _fuse_call()(
        parts_sp[0][:, :N], parts_sp[1][:, :N],
        parts_ge[0][:, :N], parts_ge[1][:, :N], pad8(b_sp), pad8(b_ge))

    h3, as3, ads3 = _proj_call(1, OUT_CH, HEADS * HID)(
        h_fused, W_fu, att8(att_src_fu), att8(att_dst_fu))
    _, parts3 = _edge_layer(
        src_sp, dst_sp, h3, _pad_rows(as3, NP), _pad_rows(ads3, NP),
        1, OUT_CH, zeros_ad, zeros_h)

    emb, recon = _dec_call()(
        parts3[0][:, :N], pad8(b_fu), W_d1, pad8(b_d1), W_d2, pad8(b_d2))

    a_sp = alpha_sp[:E2, :HEADS]
    a_ge = alpha_ge[:E2, :HEADS]
    return (emb, recon, a_sp, a_ge)


# speedup vs baseline: 1.0111x; 1.0111x over previous
import functools

import jax
import jax.numpy as jnp
from jax import lax
from jax.experimental import pallas as pl
from jax.experimental.pallas import tpu as pltpu
from jax.experimental.pallas import tpu_sc as plsc

N = 10000
E = 320000
E2 = E + N
IN_CH = 128
HID = 64
OUT_CH = 64
HEADS = 4
ALPHA = 0.5
AD = 8

NC = 2
NS = 16
NW = NC * NS
K = 128
E2P = ((E2 + NW * K - 1) // (NW * K)) * (NW * K)
EPW = E2P // NW
NG = EPW // K
NP = N + 16


def _sc_gather(d):
    mesh = plsc.VectorSubcoreMesh(core_axis_name="c", subcore_axis_name="s")

    @functools.partial(
        pl.kernel, mesh=mesh,
        out_type=jax.ShapeDtypeStruct((E2P, d), jnp.float32),
        scratch_types=[
            pltpu.VMEM((K,), jnp.int32),
            pltpu.VMEM((K, d), jnp.float32),
            pltpu.SemaphoreType.DMA,
        ],
        compiler_params=pltpu.CompilerParams(use_tc_tiling_on_sc=False),
    )
    def k(table_hbm, idx_hbm, out_hbm, idx_v, rows_v, sem):
        wid = lax.axis_index("s") * NC + lax.axis_index("c")

        def body(g, carry):
            base = wid * EPW + g * K
            pltpu.sync_copy(idx_hbm.at[pl.ds(base, K)], idx_v)
            pltpu.async_copy(table_hbm.at[idx_v], rows_v, sem).wait()
            pltpu.sync_copy(rows_v, out_hbm.at[pl.ds(base, K)])
            return carry

        lax.fori_loop(0, NG, body, 0)

    return k


def _sc_scatter_add(d):
    mesh = plsc.VectorSubcoreMesh(core_axis_name="c", subcore_axis_name="s")
    rpt = NP // NS

    @functools.partial(
        pl.kernel, mesh=mesh,
        out_type=jax.ShapeDtypeStruct((NC, NP, d), jnp.float32),
        scratch_types=[
            pltpu.VMEM((K,), jnp.int32),
            pltpu.VMEM((K, d), jnp.float32),
            pltpu.VMEM_SHARED((NP, d), jnp.float32),
        ],
        compiler_params=pltpu.CompilerParams(use_tc_tiling_on_sc=False),
    )
    def k(vals_hbm, idx_hbm, zeros_hbm, out_hbm, idx_v, vals_v, acc_sh):
        cid = lax.axis_index("c")
        sid = lax.axis_index("s")
        wid = sid * NC + cid
        pltpu.sync_copy(zeros_hbm.at[pl.ds(sid * rpt, rpt)],
                        acc_sh.at[pl.ds(sid * rpt, rpt)])
        plsc.subcore_barrier()

        def body(g, carry):
            base = wid * EPW + g * K
            pltpu.sync_copy(idx_hbm.at[pl.ds(base, K)], idx_v)
            pltpu.sync_copy(vals_hbm.at[pl.ds(base, K)], vals_v)
            pltpu.sync_copy(vals_v, acc_sh.at[idx_v], add=True)
            return carry

        lax.fori_loop(0, NG, body, 0)
        plsc.subcore_barrier()
        pltpu.sync_copy(acc_sh.at[pl.ds(sid * rpt, rpt)],
                        out_hbm.at[cid, pl.ds(sid * rpt, rpt)])

    return k


def _elu(v):
    return jnp.where(v > 0, v, jnp.exp(jnp.minimum(v, 0.0)) - 1.0)

def _proj_call(heads, hid, din):
    bn = 400
    w = heads * hid

    def body(x_ref, w_ref, asrc_ref, adst_ref, h_ref, as_ref, ad_ref):
        h = jnp.dot(x_ref[...], w_ref[...], preferred_element_type=jnp.float32)
        h_ref[...] = h
        h4 = h.reshape(bn, heads, hid)
        a_s = (h4 * asrc_ref[...][:heads][None]).sum(-1)
        a_d = (h4 * adst_ref[...][:heads][None]).sum(-1)
        pad = jnp.zeros((bn, AD - heads), jnp.float32)
        as_ref[...] = jnp.concatenate([a_s, pad], axis=1)
        ad_ref[...] = jnp.concatenate([a_d, pad], axis=1)

    return pl.pallas_call(
        body,
        grid=(N // bn,),
        in_specs=[
            pl.BlockSpec((bn, din), lambda i: (i, 0)),
            pl.BlockSpec((din, w), lambda i: (0, 0)),
            pl.BlockSpec((8, hid), lambda i: (0, 0)),
            pl.BlockSpec((8, hid), lambda i: (0, 0)),
        ],
        out_specs=[
            pl.BlockSpec((bn, w), lambda i: (i, 0)),
            pl.BlockSpec((bn, AD), lambda i: (i, 0)),
            pl.BlockSpec((bn, AD), lambda i: (i, 0)),
        ],
        out_shape=[
            jax.ShapeDtypeStruct((N, w), jnp.float32),
            jax.ShapeDtypeStruct((N, AD), jnp.float32),
            jax.ShapeDtypeStruct((N, AD), jnp.float32),
        ],
    )


_EW_R = E2P * AD // 128


def _eexp_call():
    br = 256

    def body(s_ref, d_ref, o_ref):
        v = s_ref[...] + d_ref[...]
        v = jnp.where(v >= 0, v, 0.2 * v)
        o_ref[...] = jnp.exp(v)

    return pl.pallas_call(
        body,
        grid=(_EW_R // br,),
        in_specs=[pl.BlockSpec((br, 128), lambda i: (i, 0))] * 2,
        out_specs=pl.BlockSpec((br, 128), lambda i: (i, 0)),
        out_shape=jax.ShapeDtypeStruct((_EW_R, 128), jnp.float32),
    )


def _alpha_call():
    br = 256

    def body(e_ref, dn_ref, o_ref):
        o_ref[...] = e_ref[...] / (dn_ref[...] + 1e-16)

    return pl.pallas_call(
        body,
        grid=(_EW_R // br,),
        in_specs=[pl.BlockSpec((br, 128), lambda i: (i, 0))] * 2,
        out_specs=pl.BlockSpec((br, 128), lambda i: (i, 0)),
        out_shape=jax.ShapeDtypeStruct((_EW_R, 128), jnp.float32),
    )


def _densum_call(d):
    def body(p_ref, o_ref):
        o_ref[...] = p_ref[0] + p_ref[1]

    return pl.pallas_call(
        body,
        grid=(1,),
        in_specs=[pl.BlockSpec((2, NP, d), lambda i: (0, 0, 0))],
        out_specs=pl.BlockSpec((NP, d), lambda i: (0, 0)),
        out_shape=jax.ShapeDtypeStruct((NP, d), jnp.float32),
    )


def _msg_call(heads, hid):
    bm = 512
    w = heads * hid
    nout = max(1, w // 128)
    wo = w // nout

    def body(h_ref, a_ref, *o_refs):
        hh = h_ref[...].reshape(bm, heads, hid)
        al = a_ref[...][:, :heads]
        m = (hh * al[:, :, None]).reshape(bm, w)
        for j, o_ref in enumerate(o_refs):
            o_ref[...] = m[:, j * wo:(j + 1) * wo]

    return pl.pallas_call(
        body,
        grid=(E2P // bm,),
        in_specs=[
            pl.BlockSpec((bm, w), lambda i: (i, 0)),
            pl.BlockSpec((bm, AD), lambda i: (i, 0)),
        ],
        out_specs=[pl.BlockSpec((bm, wo), lambda i: (i, 0))] * nout,
        out_shape=[jax.ShapeDtypeStruct((E2P, wo), jnp.float32)] * nout,
    )


def _fuse_call():
    bn = 400

    def body(sp0_ref, sp1_ref, ge0_ref, ge1_ref, bsp_ref, bge_ref, o_ref):
        sp = jnp.concatenate([sp0_ref[0] + sp0_ref[1],
                              sp1_ref[0] + sp1_ref[1]], axis=1)
        ge = jnp.concatenate([ge0_ref[0] + ge0_ref[1],
                              ge1_ref[0] + ge1_ref[1]], axis=1)
        sp = _elu(sp + bsp_ref[...][0][None])
        ge = _elu(ge + bge_ref[...][0][None])
        o_ref[...] = ALPHA * ge + (1.0 - ALPHA) * sp

    w = HEADS * HID
    return pl.pallas_call(
        body,
        grid=(N // bn,),
        in_specs=[pl.BlockSpec((2, bn, 128), lambda i: (0, i, 0))] * 4
        + [pl.BlockSpec((8, w), lambda i: (0, 0))] * 2,
        out_specs=pl.BlockSpec((bn, w), lambda i: (i, 0)),
        out_shape=jax.ShapeDtypeStruct((N, w), jnp.float32),
    )


def _dec_call():
    bn = 400

    def body(p_ref, bfu_ref, w1_ref, b1_ref, w2_ref, b2_ref,
             emb_ref, rec_ref):
        emb = p_ref[0] + p_ref[1] + bfu_ref[...][0][None]
        emb_ref[...] = emb
        d1 = _elu(
            jnp.dot(emb, w1_ref[...], preferred_element_type=jnp.float32)
            + b1_ref[...][0][None])
        rec_ref[...] = jax.nn.relu(
            jnp.dot(d1, w2_ref[...], preferred_element_type=jnp.float32)
            + b2_ref[...][0][None])

    return pl.pallas_call(
        body,
        grid=(N // bn,),
        in_specs=[
            pl.BlockSpec((2, bn, OUT_CH), lambda i: (0, i, 0)),
            pl.BlockSpec((8, OUT_CH), lambda i: (0, 0)),
            pl.BlockSpec((OUT_CH, HID), lambda i: (0, 0)),
            pl.BlockSpec((8, HID), lambda i: (0, 0)),
            pl.BlockSpec((HID, IN_CH), lambda i: (0, 0)),
            pl.BlockSpec((8, IN_CH), lambda i: (0, 0)),
        ],
        out_specs=[
            pl.BlockSpec((bn, OUT_CH), lambda i: (i, 0)),
            pl.BlockSpec((bn, IN_CH), lambda i: (i, 0)),
        ],
        out_shape=[
            jax.ShapeDtypeStruct((N, OUT_CH), jnp.float32),
            jax.ShapeDtypeStruct((N, IN_CH), jnp.float32),
        ],
    )


def _pad_rows(a, rows):
    return jnp.concatenate(
        [a, jnp.zeros((rows - a.shape[0],) + a.shape[1:], a.dtype)], axis=0)


def _edge_layer(src, dst, h, a_src_p, a_dst_p, heads, hid, zeros_ad, zeros_h):
    as_e = _sc_gather(AD)(a_src_p, src)
    ad_e = _sc_gather(AD)(a_dst_p, dst)
    e_exp2 = _eexp_call()(as_e.reshape(_EW_R, 128), ad_e.reshape(_EW_R, 128))
    den_p = _sc_scatter_add(AD)(e_exp2.reshape(E2P, AD), dst, zeros_ad)
    den = _densum_call(AD)(den_p)
    dn_e = _sc_gather(AD)(den, dst)
    alpha2 = _alpha_call()(e_exp2, dn_e.reshape(_EW_R, 128))
    alpha_p = alpha2.reshape(E2P, AD)
    h_e = _sc_gather(heads * hid)(h, src)
    msgs = _msg_call(heads, hid)(h_e, alpha_p)
    if not isinstance(msgs, (list, tuple)):
        msgs = [msgs]
    parts = [_sc_scatter_add(m.shape[1])(m, dst, zeros_h[:, :m.shape[1]])
             for m in msgs]
    return alpha_p, parts


def kernel(x, spatial_edge_index, gene_sim_edge_index, W_sp, att_src_sp,
           att_dst_sp, b_sp, W_ge, att_src_ge, att_dst_ge, b_ge, W_fu,
           att_src_fu, att_dst_fu, b_fu, W_d1, b_d1, W_d2, b_d2):
    i32 = jnp.int32
    loop = jnp.arange(N, dtype=i32)
    pad_n = E2P - E2

    def mk_edges(ei):
        src = jnp.concatenate(
            [ei[0].astype(i32), loop, jnp.zeros((pad_n,), i32)])
        dst = jnp.concatenate(
            [ei[1].astype(i32), loop, jnp.full((pad_n,), N, i32)])
        return src, dst

    src_sp, dst_sp = mk_edges(spatial_edge_index)
    src_ge, dst_ge = mk_edges(gene_sim_edge_index)

    zeros_ad = jnp.zeros((NP, AD), jnp.float32)
    zeros_h = jnp.zeros((NP, 128), jnp.float32)

    def pad8(b):
        return jnp.tile(b.reshape(1, -1), (8, 1))

    att8 = lambda a: _pad_rows(a, 8)

    h_sp, as_sp, ads_sp = _proj_call(HEADS, HID, IN_CH)(
        x, W_sp, att8(att_src_sp), att8(att_dst_sp))
    h_ge, as_ge, ads_ge = _proj_call(HEADS, HID, IN_CH)(
        x, W_ge, att8(att_src_ge), att8(att_dst_ge))

    alpha_sp, parts_sp = _edge_layer(
        src_sp, dst_sp, h_sp, _pad_rows(as_sp, NP), _pad_rows(ads_sp, NP),
        HEADS, HID, zeros_ad, zeros_h)
    alpha_ge, parts_ge = _edge_layer(
        src_ge, dst_ge, h_ge, _pad_rows(as_ge, NP), _pad_rows(ads_ge, NP),
        HEADS, HID, zeros_ad, zeros_h)

    h_fused = _fuse_call()(
        parts_sp[0][:, :N], parts_sp[1][:, :N],
        parts_ge[0][:, :N], parts_ge[1][:, :N], pad8(b_sp), pad8(b_ge))

    h3, as3, ads3 = _proj_call(1, OUT_CH, HEADS * HID)(
        h_fused, W_fu, att8(att_src_fu), att8(att_dst_fu))
    _, parts3 = _edge_layer(
        src_sp, dst_sp, h3, _pad_rows(as3, NP), _pad_rows(ads3, NP),
        1, OUT_CH, zeros_ad, zeros_h)

    emb, recon = _dec_call()(
        parts3[0][:, :N], pad8(b_fu), W_d1, pad8(b_d1), W_d2, pad8(b_d2))

    a_sp = alpha_sp[:E2, :HEADS]
    a_ge = alpha_ge[:E2, :HEADS]
    return (emb, recon, a_sp, a_ge)
